# baseline (speedup 1.0000x reference)
import jax
import jax.numpy as jnp
import numpy as np
from jax.experimental import pallas as pl


def _gcn_conv(x, edge_index, W, b):
    n = x.shape[0]
    x = x @ W
    loops = jnp.arange(n, dtype=edge_index.dtype)
    src = jnp.concatenate([edge_index[0], loops])
    dst = jnp.concatenate([edge_index[1], loops])
    ew = jnp.ones(src.shape[0], dtype=x.dtype)
    deg = jnp.zeros((n,), dtype=x.dtype).at[dst].add(ew)
    dis = jnp.where(deg > 0, jax.lax.rsqrt(jnp.maximum(deg, 1e-12)), 0.0)
    norm = dis[src] * dis[dst]
    msg = x[src] * norm[:, None]
    out = jax.ops.segment_sum(msg, dst, num_segments=n)
    return out + b


def kernel(x_raw, sincos_pe, edge_index, batch, W1, b1, Wmu, bmu, Wlv, blv, pool_w, eps):
    edge_index = edge_index[:, ::-1]
    x_in = jnp.concatenate([x_raw, sincos_pe], axis=-1)
    h = jax.nn.relu(_gcn_conv(x_in, edge_index, W1, b1))
    mu = _gcn_conv(h, edge_index, Wmu, bmu)
    logvar = _gcn_conv(h, edge_index, Wlv, blv)
    std = jnp.exp(0.5 * logvar)
    z = mu + eps * std
    score = (z * pool_w).sum(axis=-1) / jnp.linalg.norm(pool_w)
    score = jnp.tanh(score)
    k = int(np.ceil(0.5 * z.shape[0]))
    _, perm = jax.lax.top_k(score, k)
    z_pool = z[perm] * score[perm][:, None]
    mu_pool = mu[perm]
    logvar_pool = logvar[perm]
    batch_pool = batch[perm]
    return (z_pool, mu_pool, logvar_pool, batch_pool, perm)


# speedup vs baseline: 1.0068x; 1.0068x over previous
import functools

import jax
import jax.numpy as jnp
import numpy as np
from jax.experimental import pallas as pl
from jax.experimental.pallas import tpu as pltpu
from jax.experimental.pallas import tpu_sc as plsc


def _mm_body(x_ref, w_ref, o_ref):
    o_ref[...] = jnp.dot(x_ref[...], w_ref[...],
                         preferred_element_type=jnp.float32)


def _pallas_mm(x, w):
    m, k = x.shape
    k2, n = w.shape
    bm = 2000
    return pl.pallas_call(
        _mm_body,
        grid=(m // bm,),
        in_specs=[pl.BlockSpec((bm, k), lambda i: (i, 0)),
                  pl.BlockSpec((k2, n), lambda i: (0, 0))],
        out_specs=pl.BlockSpec((bm, n), lambda i: (i, 0)),
        out_shape=jax.ShapeDtypeStruct((m, n), jnp.float32),
    )(x, w)


def _mult_body(xr_ref, ds_ref, dd_ref, msg_ref, norm_ref):
    norm = ds_ref[:, 0:1] * dd_ref[:, 0:1]
    norm_ref[...] = norm
    msg_ref[...] = xr_ref[...] * norm


def _pallas_mult(xr, ds16, dd16):
    e, d = xr.shape
    bm = 4000
    return pl.pallas_call(
        _mult_body,
        grid=(e // bm,),
        in_specs=[pl.BlockSpec((bm, d), lambda i: (i, 0)),
                  pl.BlockSpec((bm, 16), lambda i: (i, 0)),
                  pl.BlockSpec((bm, 16), lambda i: (i, 0))],
        out_specs=[pl.BlockSpec((bm, d), lambda i: (i, 0)),
                   pl.BlockSpec((bm, 1), lambda i: (i, 0))],
        out_shape=[jax.ShapeDtypeStruct((e, d), jnp.float32),
                   jax.ShapeDtypeStruct((e, 1), jnp.float32)],
    )(xr, ds16, dd16)


def _mult2_body(xr_ref, norm_ref, msg_ref):
    msg_ref[...] = xr_ref[...] * norm_ref[...]


def _pallas_mult2(xr, norm):
    e, d = xr.shape
    bm = 4000
    return pl.pallas_call(
        _mult2_body,
        grid=(e // bm,),
        in_specs=[pl.BlockSpec((bm, d), lambda i: (i, 0)),
                  pl.BlockSpec((bm, 1), lambda i: (i, 0))],
        out_specs=pl.BlockSpec((bm, d), lambda i: (i, 0)),
        out_shape=jax.ShapeDtypeStruct((e, d), jnp.float32),
    )(xr, norm)


def _self_body(y_ref, dis_ref, o_ref):
    nii = dis_ref[...] * dis_ref[...]
    o_ref[...] = y_ref[...] * nii


def _pallas_selfmul(y, dis):
    n, d = y.shape
    bm = 2000
    return pl.pallas_call(
        _self_body,
        grid=(n // bm,),
        in_specs=[pl.BlockSpec((bm, d), lambda i: (i, 0)),
                  pl.BlockSpec((bm, 1), lambda i: (i, 0))],
        out_specs=pl.BlockSpec((bm, d), lambda i: (i, 0)),
        out_shape=jax.ShapeDtypeStruct((n, d), jnp.float32),
    )(y, dis[:, None])


def _post_body(mu_ref, lv_ref, eps_ref, w_ref, z_ref, zw_ref):
    std = jnp.exp(0.5 * lv_ref[...])
    z = mu_ref[...] + eps_ref[...] * std
    z_ref[...] = z
    zw_ref[...] = z * w_ref[0:1, :]


def _pallas_post(mu, lv, eps, pool_w):
    n, d = mu.shape
    bm = 2000
    w2d = jnp.broadcast_to(pool_w[None, :], (8, d))
    return pl.pallas_call(
        _post_body,
        grid=(n // bm,),
        in_specs=[pl.BlockSpec((bm, d), lambda i: (i, 0)),
                  pl.BlockSpec((bm, d), lambda i: (i, 0)),
                  pl.BlockSpec((bm, d), lambda i: (i, 0)),
                  pl.BlockSpec((8, d), lambda i: (0, 0))],
        out_specs=[pl.BlockSpec((bm, d), lambda i: (i, 0)),
                   pl.BlockSpec((bm, d), lambda i: (i, 0))],
        out_shape=[jax.ShapeDtypeStruct((n, d), jnp.float32),
                   jax.ShapeDtypeStruct((n, d), jnp.float32)],
    )(mu, lv, eps, w2d)


_W = 128


def _sc_gather3(x, dis16, src, dst):
    e = src.shape[0]
    n, d = x.shape
    src2 = src.reshape(1, e)
    dst2 = dst.reshape(1, e)
    mesh = plsc.VectorSubcoreMesh(core_axis_name="c", subcore_axis_name="s")

    @functools.partial(
        pl.kernel, mesh=mesh,
        out_type=[jax.ShapeDtypeStruct((e, d), jnp.float32),
                  jax.ShapeDtypeStruct((e, 16), jnp.float32),
                  jax.ShapeDtypeStruct((e, 16), jnp.float32)])
    def k(x_hbm, d16_hbm, s_hbm, t_hbm, o1_hbm, o2_hbm, o3_hbm):
        def body(s_v, t_v, o1, o2, o3):
            pltpu.sync_copy(x_hbm.at[s_v.at[0]], o1)
            pltpu.sync_copy(d16_hbm.at[s_v.at[0]], o2)
            pltpu.sync_copy(d16_hbm.at[t_v.at[0]], o3)

        pltpu.emit_pipeline(
            body,
            grid=(e // _W,),
            in_specs=[pl.BlockSpec((1, _W), lambda i: (0, i)),
                      pl.BlockSpec((1, _W), lambda i: (0, i))],
            out_specs=[pl.BlockSpec((_W, d), lambda i: (i, 0)),
                       pl.BlockSpec((_W, 16), lambda i: (i, 0)),
                       pl.BlockSpec((_W, 16), lambda i: (i, 0))],
            core_axis_name=("c", "s"),
            dimension_semantics=(pltpu.PARALLEL,),
        )(s_hbm, t_hbm, o1_hbm, o2_hbm, o3_hbm)

    return k(x, dis16, src2, dst2)


def _sc_gather1(x, src):
    e = src.shape[0]
    n, d = x.shape
    src2 = src.reshape(1, e)
    mesh = plsc.VectorSubcoreMesh(core_axis_name="c", subcore_axis_name="s")

    @functools.partial(
        pl.kernel, mesh=mesh,
        out_type=jax.ShapeDtypeStruct((e, d), jnp.float32))
    def k(x_hbm, s_hbm, o_hbm):
        def body(s_v, o1):
            pltpu.sync_copy(x_hbm.at[s_v.at[0]], o1)

        pltpu.emit_pipeline(
            body,
            grid=(e // _W,),
            in_specs=[pl.BlockSpec((1, _W), lambda i: (0, i))],
            out_specs=[pl.BlockSpec((_W, d), lambda i: (i, 0))],
            core_axis_name=("c", "s"),
            dimension_semantics=(pltpu.PARALLEL,),
        )(s_hbm, o_hbm)

    return k(x, src2)


def kernel(x_raw, sincos_pe, edge_index, batch, W1, b1, Wmu, bmu, Wlv, blv, pool_w, eps):
    n = x_raw.shape[0]
    x_in = jnp.concatenate([x_raw, sincos_pe], axis=-1)
    loops = jnp.arange(n, dtype=edge_index.dtype)
    dst_full = jnp.concatenate([edge_index[1], loops])
    src32 = edge_index[0].astype(jnp.int32)
    dst32 = edge_index[1].astype(jnp.int32)

    ew = jnp.ones(dst_full.shape[0], dtype=jnp.float32)
    deg = jnp.zeros((n,), jnp.float32).at[dst_full].add(ew)
    dis = jnp.where(deg > 0, jax.lax.rsqrt(jnp.maximum(deg, 1e-12)), 0.0)
    dis16 = jnp.broadcast_to(dis[:, None], (n, 16))

    y1 = _pallas_mm(x_in, W1)
    xr = _sc_gather1(y1, src32)
    norm_e = (dis[src32] * dis[dst32])[:, None]
    msg1 = _pallas_mult2(xr, norm_e)
    self1 = _pallas_selfmul(y1, dis)
    upd1 = jnp.concatenate([msg1, self1], axis=0)
    s1 = jax.ops.segment_sum(upd1, dst_full, num_segments=n)
    h = jax.nn.relu(s1 + b1)

    hc = _pallas_mm(h, jnp.concatenate([Wmu, Wlv], axis=1))
    hr = _sc_gather1(hc, src32)
    msg2 = _pallas_mult2(hr, norm_e)
    self2 = _pallas_selfmul(hc, dis)
    mu = jax.ops.segment_sum(
        jnp.concatenate([msg2[:, :64], self2[:, :64]], axis=0),
        dst_full, num_segments=n) + bmu
    logvar = jax.ops.segment_sum(
        jnp.concatenate([msg2[:, 64:], self2[:, 64:]], axis=0),
        dst_full, num_segments=n) + blv

    z, zw = _pallas_post(mu, logvar, eps, pool_w)
    score = jnp.tanh(zw.sum(axis=-1) / jnp.linalg.norm(pool_w))
    k = int(np.ceil(0.5 * n))
    _, perm = jax.lax.top_k(score, k)
    z_pool = z[perm] * score[perm][:, None]
    mu_pool = mu[perm]
    logvar_pool = logvar[perm]
    batch_pool = batch[perm]
    return (z_pool, mu_pool, logvar_pool, batch_pool, perm)


# speedup vs baseline: 1.0068x; 1.0000x over previous
import functools

import jax
import jax.numpy as jnp
import numpy as np
from jax.experimental import pallas as pl
from jax.experimental.pallas import tpu as pltpu
from jax.experimental.pallas import tpu_sc as plsc


def _mm_body(x_ref, w_ref, o_ref):
    o_ref[...] = jnp.dot(x_ref[...], w_ref[...],
                         preferred_element_type=jnp.float32)


def _pallas_mm(x, w):
    m, k = x.shape
    k2, n = w.shape
    bm = 2000
    return pl.pallas_call(
        _mm_body,
        grid=(m // bm,),
        in_specs=[pl.BlockSpec((bm, k), lambda i: (i, 0)),
                  pl.BlockSpec((k2, n), lambda i: (0, 0))],
        out_specs=pl.BlockSpec((bm, n), lambda i: (i, 0)),
        out_shape=jax.ShapeDtypeStruct((m, n), jnp.float32),
    )(x, w)


def _mult2_body(xr_ref, norm_ref, msg_ref):
    msg_ref[...] = xr_ref[...] * norm_ref[...]


def _pallas_mult2(xr, norm):
    e, d = xr.shape
    bm = 4000
    return pl.pallas_call(
        _mult2_body,
        grid=(e // bm,),
        in_specs=[pl.BlockSpec((bm, d), lambda i: (i, 0)),
                  pl.BlockSpec((bm, 1), lambda i: (i, 0))],
        out_specs=pl.BlockSpec((bm, d), lambda i: (i, 0)),
        out_shape=jax.ShapeDtypeStruct((e, d), jnp.float32),
    )(xr, norm)


def _self_body(y_ref, dis_ref, o_ref):
    nii = dis_ref[...] * dis_ref[...]
    o_ref[...] = y_ref[...] * nii


def _pallas_selfmul(y, dis):
    n, d = y.shape
    bm = 2000
    return pl.pallas_call(
        _self_body,
        grid=(n // bm,),
        in_specs=[pl.BlockSpec((bm, d), lambda i: (i, 0)),
                  pl.BlockSpec((bm, 1), lambda i: (i, 0))],
        out_specs=pl.BlockSpec((bm, d), lambda i: (i, 0)),
        out_shape=jax.ShapeDtypeStruct((n, d), jnp.float32),
    )(y, dis[:, None])


def _post_body(mu_ref, lv_ref, eps_ref, w_ref, z_ref, zw_ref):
    std = jnp.exp(0.5 * lv_ref[...])
    z = mu_ref[...] + eps_ref[...] * std
    z_ref[...] = z
    zw_ref[...] = z * w_ref[0:1, :]


def _pallas_post(mu, lv, eps, pool_w):
    n, d = mu.shape
    bm = 2000
    w2d = jnp.broadcast_to(pool_w[None, :], (8, d))
    return pl.pallas_call(
        _post_body,
        grid=(n // bm,),
        in_specs=[pl.BlockSpec((bm, d), lambda i: (i, 0)),
                  pl.BlockSpec((bm, d), lambda i: (i, 0)),
                  pl.BlockSpec((bm, d), lambda i: (i, 0)),
                  pl.BlockSpec((8, d), lambda i: (0, 0))],
        out_specs=[pl.BlockSpec((bm, d), lambda i: (i, 0)),
                   pl.BlockSpec((bm, d), lambda i: (i, 0))],
        out_shape=[jax.ShapeDtypeStruct((n, d), jnp.float32),
                   jax.ShapeDtypeStruct((n, d), jnp.float32)],
    )(mu, lv, eps, w2d)


_W = 128


def _sc_gather1(x, src):
    e = src.shape[0]
    n, d = x.shape
    src2 = src.reshape(1, e)
    mesh = plsc.VectorSubcoreMesh(core_axis_name="c", subcore_axis_name="s")

    @functools.partial(
        pl.kernel, mesh=mesh,
        out_type=jax.ShapeDtypeStruct((e, d), jnp.float32))
    def k(x_hbm, s_hbm, o_hbm):
        def body(s_v, o1):
            pltpu.sync_copy(x_hbm.at[s_v.at[0]], o1)

        pltpu.emit_pipeline(
            body,
            grid=(e // _W,),
            in_specs=[pl.BlockSpec((1, _W), lambda i: (0, i))],
            out_specs=[pl.BlockSpec((_W, d), lambda i: (i, 0))],
            core_axis_name=("c", "s"),
            dimension_semantics=(pltpu.PARALLEL,),
        )(s_hbm, o_hbm)

    return k(x, src2)


def kernel(x_raw, sincos_pe, edge_index, batch, W1, b1, Wmu, bmu, Wlv, blv, pool_w, eps):
    n = x_raw.shape[0]
    x_in = jnp.concatenate([x_raw, sincos_pe], axis=-1)
    loops = jnp.arange(n, dtype=edge_index.dtype)
    dst_full = jnp.concatenate([edge_index[1], loops])
    src32 = edge_index[0].astype(jnp.int32)
    dst32 = edge_index[1].astype(jnp.int32)

    ew = jnp.ones(dst_full.shape[0], dtype=jnp.float32)
    deg = jnp.zeros((n,), jnp.float32).at[dst_full].add(ew)
    dis = jnp.where(deg > 0, jax.lax.rsqrt(jnp.maximum(deg, 1e-12)), 0.0)

    y1 = _pallas_mm(x_in, W1)
    xr = _sc_gather1(y1, src32)
    norm_e = (dis[src32] * dis[dst32])[:, None]
    msg1 = _pallas_mult2(xr, norm_e)
    self1 = _pallas_selfmul(y1, dis)
    upd1 = jnp.concatenate([msg1, self1], axis=0)
    s1 = jax.ops.segment_sum(upd1, dst_full, num_segments=n)
    h = jax.nn.relu(s1 + b1)

    hc = _pallas_mm(h, jnp.concatenate([Wmu, Wlv], axis=1))
    hr = _sc_gather1(hc, src32)
    msg2 = _pallas_mult2(hr, norm_e)
    self2 = _pallas_selfmul(hc, dis)
    mu = jax.ops.segment_sum(
        jnp.concatenate([msg2[:, :64], self2[:, :64]], axis=0),
        dst_full, num_segments=n) + bmu
    logvar = jax.ops.segment_sum(
        jnp.concatenate([msg2[:, 64:], self2[:, 64:]], axis=0),
        dst_full, num_segments=n) + blv

    z, zw = _pallas_post(mu, logvar, eps, pool_w)
    score = jnp.tanh(zw.sum(axis=-1) / jnp.linalg.norm(pool_w))
    k = int(np.ceil(0.5 * n))
    _, perm = jax.lax.top_k(score, k)
    z_pool = z[perm] * score[perm][:, None]
    mu_pool = mu[perm]
    logvar_pool = logvar[perm]
    batch_pool = batch[perm]
    return (z_pool, mu_pool, logvar_pool, batch_pool, perm)
